# baseline (speedup 1.0000x reference)
import functools
import math

import jax
import jax.numpy as jnp
from jax.experimental import pallas as pl

B, C, H, W = 2, 32, 64, 96
N = H * W
TEMP_SCALE = 1.0 / math.sqrt(0.1)
K_POINTS = 500
K_PAD = 512
TILE_R = 512
NR = N // TILE_R


def _pass1_kernel(d0_ref, d1_ref, c0_ref, c1_ref,
                  rowmax_o, rowsum_o, rowamax_o, v0_o,
                  colmax_o, colsum_o, colamax_o, v1_o):
    r = pl.program_id(1)
    r0 = r * TILE_R
    a = jnp.float32(TEMP_SCALE)
    d0t = d0_ref[0] * a
    d1t = d1_ref[0] * a
    s = jax.lax.dot_general(d0t, d1t, (((0,), (0,)), ((), ())),
                            preferred_element_type=jnp.float32)
    col = jax.lax.broadcasted_iota(jnp.int32, (TILE_R, N), 1)
    row_l = jax.lax.broadcasted_iota(jnp.int32, (TILE_R, N), 0)

    rmax = jnp.max(s, axis=1)
    ramax = jnp.min(jnp.where(s == rmax[:, None], col, N), axis=1)
    rsum = jnp.sum(jnp.exp(s - rmax[:, None]), axis=1)
    c0t = c0_ref[0, 0, pl.ds(r0, TILE_R)]
    v0 = jnp.sum(jnp.where(col == c0t[:, None], s, 0.0), axis=1)
    rowmax_o[0, 0, pl.ds(r0, TILE_R)] = rmax
    rowamax_o[0, 0, pl.ds(r0, TILE_R)] = ramax
    rowsum_o[0, 0, pl.ds(r0, TILE_R)] = rsum
    v0_o[0, 0, pl.ds(r0, TILE_R)] = v0

    @pl.when(r == 0)
    def _init():
        colmax_o[...] = jnp.full((1, 1, N), -jnp.inf, jnp.float32)
        colsum_o[...] = jnp.zeros((1, 1, N), jnp.float32)
        colamax_o[...] = jnp.zeros((1, 1, N), jnp.int32)
        v1_o[...] = jnp.zeros((1, 1, N), jnp.float32)

    old_max = colmax_o[0, 0, :]
    old_sum = colsum_o[0, 0, :]
    old_amax = colamax_o[0, 0, :]
    tmax = jnp.max(s, axis=0)
    tamax = jnp.min(jnp.where(s == tmax[None, :], row_l, TILE_R), axis=0) + r0
    nmax = jnp.maximum(old_max, tmax)
    tsum = jnp.sum(jnp.exp(s - nmax[None, :]), axis=0)
    colmax_o[0, 0, :] = nmax
    colsum_o[0, 0, :] = old_sum * jnp.exp(old_max - nmax) + tsum
    colamax_o[0, 0, :] = jnp.where(tmax > old_max, tamax, old_amax)
    c1v = c1_ref[0, 0, :]
    hit = (c1v[None, :] - r0) == row_l
    v1_o[0, 0, :] += jnp.sum(jnp.where(hit, s, 0.0), axis=0)


def _bce(logits, targets):
    return (jnp.maximum(logits, 0.0) - logits * targets
            + jnp.log1p(jnp.exp(-jnp.abs(logits))))


def _pass2_kernel(rowmax_r, rowsum_r, rowamax_r, v0_r,
                  colmax_r, colsum_r, colamax_r, v1_r,
                  c0_r, c1_r, l0_r, l1_r,
                  desc_o, kp_o, prec_o, rec_o,
                  rows0_o, cols0_o, rows1_o, cols1_o, conf_o):
    colK = jax.lax.broadcasted_iota(jnp.int32, (K_PAD, N), 1)

    def topk_idx(p):
        colT2 = jax.lax.broadcasted_iota(jnp.int32, (TILE_R, N), 1)
        ranks = []
        for t in range(NR):
            pt = p[t * TILE_R:(t + 1) * TILE_R]
            gi = t * TILE_R + jax.lax.broadcasted_iota(
                jnp.int32, (TILE_R, N), 0)
            beat = (p[None, :] > pt[:, None]) | (
                (p[None, :] == pt[:, None]) & (colT2 < gi))
            ranks.append(jnp.sum(beat.astype(jnp.int32), axis=1))
        rank = jnp.concatenate(ranks)
        posK = jax.lax.broadcasted_iota(jnp.int32, (K_PAD, N), 0)
        return jnp.sum(jnp.where(rank[None, :] == posK, colK, 0), axis=1)

    per_b = []
    for b in range(B):
        rowmax = rowmax_r[b, 0, :]
        rowsum = rowsum_r[b, 0, :]
        rowamax = rowamax_r[b, 0, :]
        v0 = v0_r[b, 0, :]
        colmax = colmax_r[b, 0, :]
        colsum = colsum_r[b, 0, :]
        colamax = colamax_r[b, 0, :]
        v1 = v1_r[b, 0, :]
        c0 = c0_r[b, 0, :]
        c1 = c1_r[b, 0, :]
        l0 = l0_r[b, 0, :]
        l1 = l1_r[b, 0, :]

        loss_0 = jnp.mean(-(v0 - rowmax - jnp.log(rowsum)))
        loss_1 = jnp.mean(-(v1 - colmax - jnp.log(colsum)))
        desc_loss = 0.5 * (loss_0 + loss_1)

        cm0 = rowamax == c0
        cm1 = colamax == c1
        colT = jax.lax.broadcasted_iota(jnp.int32, (TILE_R, N), 1)
        muts = []
        for t in range(NR):
            ra = rowamax[t * TILE_R:(t + 1) * TILE_R]
            gi = t * TILE_R + jax.lax.broadcasted_iota(
                jnp.int32, (TILE_R, N), 0)
            m = (ra[:, None] == colT) & (colamax[None, :] == gi)
            muts.append(jnp.any(m, axis=1).astype(jnp.int32))
        mutual = jnp.concatenate(muts) > 0
        n_correct = jnp.sum((cm0 & mutual).astype(jnp.int32))
        precision = n_correct / jnp.maximum(
            jnp.sum(mutual.astype(jnp.int32)), 1)
        recall = jnp.sum(cm0.astype(jnp.int32)) / N
        kp_loss = 0.5 * (jnp.mean(_bce(l0, cm0.astype(jnp.float32)))
                         + jnp.mean(_bce(l1, cm1.astype(jnp.float32))))

        sim0 = 1.0 / rowsum
        sim1 = 1.0 / colsum
        keep = cm0 & cm1
        p0 = jnp.where(keep, jax.nn.sigmoid(l0), 0.0)
        p1 = jnp.where(keep, jax.nn.sigmoid(l1), 0.0)
        idx0 = topk_idx(p0)
        idx1 = topk_idx(p1)
        f0 = idx0.astype(jnp.float32)
        f1 = idx1.astype(jnp.float32)
        r0f = jnp.floor(f0 / W)
        r1f = jnp.floor(f1 / W)
        rows0_o[b, :] = r0f
        cols0_o[b, :] = f0 - r0f * W
        rows1_o[b, :] = r1f
        cols1_o[b, :] = f1 - r1f * W
        per_b.append(dict(desc=desc_loss, kp=kp_loss, prec=precision,
                          rec=recall, sim0=sim0, sim1=sim1,
                          idx0=idx0, idx1=idx1))

    def store_scalar(ref, x):
        ref[...] = jnp.broadcast_to(x, (1, 1))

    store_scalar(desc_o, 0.5 * (per_b[0]["desc"] + per_b[1]["desc"]))
    store_scalar(kp_o, 0.5 * (per_b[0]["kp"] + per_b[1]["kp"]))
    store_scalar(prec_o, 0.5 * (per_b[0]["prec"] + per_b[1]["prec"]))
    store_scalar(rec_o, 0.5 * (per_b[0]["rec"] + per_b[1]["rec"]))

    for b1 in range(B):
        for b2 in range(B):
            g0 = jnp.sum(jnp.where(per_b[b2]["idx0"][:, None] == colK,
                                   per_b[b1]["sim0"][None, :], 0.0), axis=1)
            g1 = jnp.sum(jnp.where(per_b[b2]["idx1"][:, None] == colK,
                                   per_b[b1]["sim1"][None, :], 0.0), axis=1)
            conf_o[b1, b2, :] = g0 * g1


@jax.jit
def kernel(desc_0, desc_1, corr_0, corr_1, logits_0, logits_1):
    d0 = desc_0.reshape(B, C, N)
    d1 = desc_1.reshape(B, C, N)
    c0 = corr_0.reshape(B, 1, N)
    c1 = corr_1.reshape(B, 1, N)
    l0 = logits_0.reshape(B, 1, N)
    l1 = logits_1.reshape(B, 1, N)

    stat = jax.ShapeDtypeStruct((B, 1, N), jnp.float32)
    stat_i = jax.ShapeDtypeStruct((B, 1, N), jnp.int32)
    full = lambda shape: pl.BlockSpec(shape, lambda b, r: (b, 0, 0))
    stats = pl.pallas_call(
        _pass1_kernel,
        grid=(B, NR),
        in_specs=[
            pl.BlockSpec((1, C, TILE_R), lambda b, r: (b, 0, r)),
            pl.BlockSpec((1, C, N), lambda b, r: (b, 0, 0)),
            full((1, 1, N)),
            full((1, 1, N)),
        ],
        out_specs=[full((1, 1, N))] * 8,
        out_shape=[stat, stat, stat_i, stat,
                   stat, stat, stat_i, stat],
    )(d0, d1, c0, c1)

    scal = jax.ShapeDtypeStruct((1, 1), jnp.float32)
    kvec = jax.ShapeDtypeStruct((B, K_PAD), jnp.float32)
    outs = pl.pallas_call(
        _pass2_kernel,
        out_shape=[scal, scal, scal, scal,
                   kvec, kvec, kvec, kvec,
                   jax.ShapeDtypeStruct((B, B, K_PAD), jnp.float32)],
    )(*stats, c0, c1, l0, l1)
    desc_s, kp_s, prec_s, rec_s, rows0, cols0, rows1, cols1, conf = outs

    mp0 = jnp.stack([rows0[:, :K_POINTS], cols0[:, :K_POINTS]], axis=-1)
    mp1 = jnp.stack([rows1[:, :K_POINTS], cols1[:, :K_POINTS]], axis=-1)
    confidence = conf[None, :, :, :K_POINTS]
    return (desc_s[0, 0], kp_s[0, 0], prec_s[0, 0], rec_s[0, 0],
            mp0, mp1, confidence)


# speedup vs baseline: 1.0210x; 1.0210x over previous
import functools
import math

import jax
import jax.numpy as jnp
from jax.experimental import pallas as pl

B, C, H, W = 2, 32, 64, 96
N = H * W
TEMP_SCALE = 1.0 / math.sqrt(0.1)
K_POINTS = 500
K_PAD = 512
TILE_R = 512
NR = N // TILE_R


def _pass1_kernel(d0_ref, d1_ref, c0_ref, c1_ref,
                  rowmax_o, rowsum_o, rowamax_o, v0_o,
                  colmax_o, colsum_o, colamax_o, v1_o):
    r = pl.program_id(1)
    r0 = r * TILE_R
    a = jnp.float32(TEMP_SCALE)
    d0t = d0_ref[0] * a
    d1t = d1_ref[0] * a
    s = jax.lax.dot_general(d0t, d1t, (((0,), (0,)), ((), ())),
                            preferred_element_type=jnp.float32)
    col = jax.lax.broadcasted_iota(jnp.int32, (TILE_R, N), 1)
    row_l = jax.lax.broadcasted_iota(jnp.int32, (TILE_R, N), 0)

    rmax = jnp.max(s, axis=1)
    ramax = jnp.min(jnp.where(s == rmax[:, None], col, N), axis=1)
    rsum = jnp.sum(jnp.exp(s - rmax[:, None]), axis=1)
    c0t = c0_ref[0, 0, pl.ds(r0, TILE_R)]
    v0 = jnp.sum(jnp.where(col == c0t[:, None], s, 0.0), axis=1)
    rowmax_o[0, 0, pl.ds(r0, TILE_R)] = rmax
    rowamax_o[0, 0, pl.ds(r0, TILE_R)] = ramax
    rowsum_o[0, 0, pl.ds(r0, TILE_R)] = rsum
    v0_o[0, 0, pl.ds(r0, TILE_R)] = v0

    @pl.when(r == 0)
    def _init():
        colmax_o[...] = jnp.full((1, 1, N), -jnp.inf, jnp.float32)
        colsum_o[...] = jnp.zeros((1, 1, N), jnp.float32)
        colamax_o[...] = jnp.zeros((1, 1, N), jnp.int32)
        v1_o[...] = jnp.zeros((1, 1, N), jnp.float32)

    old_max = colmax_o[0, 0, :]
    old_sum = colsum_o[0, 0, :]
    old_amax = colamax_o[0, 0, :]
    tmax = jnp.max(s, axis=0)
    tamax = jnp.min(jnp.where(s == tmax[None, :], row_l, TILE_R), axis=0) + r0
    nmax = jnp.maximum(old_max, tmax)
    tsum = jnp.sum(jnp.exp(s - nmax[None, :]), axis=0)
    colmax_o[0, 0, :] = nmax
    colsum_o[0, 0, :] = old_sum * jnp.exp(old_max - nmax) + tsum
    colamax_o[0, 0, :] = jnp.where(tmax > old_max, tamax, old_amax)
    c1v = c1_ref[0, 0, :]
    hit = (c1v[None, :] - r0) == row_l
    v1_o[0, 0, :] += jnp.sum(jnp.where(hit, s, 0.0), axis=0)


def _bce(logits, targets):
    return (jnp.maximum(logits, 0.0) - logits * targets
            + jnp.log1p(jnp.exp(-jnp.abs(logits))))


def _pass2_kernel(rowmax_r, rowsum_r, rowamax_r, v0_r,
                  colmax_r, colsum_r, colamax_r, v1_r,
                  c0_r, c1_r, l0_r, l1_r,
                  desc_o, kp_o, prec_o, rec_o,
                  rows0_o, cols0_o, rows1_o, cols1_o, conf_o):
    colK = jax.lax.broadcasted_iota(jnp.int32, (K_PAD, N), 1)

    def topk_idx(p):
        key2 = jax.lax.bitcast_convert_type(p, jnp.int32) * 2
        colT2 = jax.lax.broadcasted_iota(jnp.int32, (TILE_R, N), 1)
        ranks = []
        for t in range(NR):
            kt = key2[t * TILE_R:(t + 1) * TILE_R]
            gi = t * TILE_R + jax.lax.broadcasted_iota(
                jnp.int32, (TILE_R, N), 0)
            beat = (key2[None, :] + (colT2 < gi).astype(jnp.int32)
                    ) > kt[:, None]
            ranks.append(jnp.sum(beat.astype(jnp.int32), axis=1))
        rank = jnp.concatenate(ranks)
        posK = jax.lax.broadcasted_iota(jnp.int32, (K_PAD, N), 0)
        return jnp.sum(jnp.where(rank[None, :] == posK, colK, 0), axis=1)

    per_b = []
    for b in range(B):
        rowmax = rowmax_r[b, 0, :]
        rowsum = rowsum_r[b, 0, :]
        rowamax = rowamax_r[b, 0, :]
        v0 = v0_r[b, 0, :]
        colmax = colmax_r[b, 0, :]
        colsum = colsum_r[b, 0, :]
        colamax = colamax_r[b, 0, :]
        v1 = v1_r[b, 0, :]
        c0 = c0_r[b, 0, :]
        c1 = c1_r[b, 0, :]
        l0 = l0_r[b, 0, :]
        l1 = l1_r[b, 0, :]

        loss_0 = jnp.mean(-(v0 - rowmax - jnp.log(rowsum)))
        loss_1 = jnp.mean(-(v1 - colmax - jnp.log(colsum)))
        desc_loss = 0.5 * (loss_0 + loss_1)

        cm0 = rowamax == c0
        cm1 = colamax == c1
        colT = jax.lax.broadcasted_iota(jnp.int32, (TILE_R, N), 1)
        muts = []
        for t in range(NR):
            ra = rowamax[t * TILE_R:(t + 1) * TILE_R]
            gi = t * TILE_R + jax.lax.broadcasted_iota(
                jnp.int32, (TILE_R, N), 0)
            m = (ra[:, None] == colT) & (colamax[None, :] == gi)
            muts.append(jnp.any(m, axis=1).astype(jnp.int32))
        mutual = jnp.concatenate(muts) > 0
        n_correct = jnp.sum((cm0 & mutual).astype(jnp.int32))
        precision = n_correct / jnp.maximum(
            jnp.sum(mutual.astype(jnp.int32)), 1)
        recall = jnp.sum(cm0.astype(jnp.int32)) / N
        kp_loss = 0.5 * (jnp.mean(_bce(l0, cm0.astype(jnp.float32)))
                         + jnp.mean(_bce(l1, cm1.astype(jnp.float32))))

        sim0 = 1.0 / rowsum
        sim1 = 1.0 / colsum
        keep = cm0 & cm1
        p0 = jnp.where(keep, jax.nn.sigmoid(l0), 0.0)
        p1 = jnp.where(keep, jax.nn.sigmoid(l1), 0.0)
        idx0 = topk_idx(p0)
        idx1 = topk_idx(p1)
        f0 = idx0.astype(jnp.float32)
        f1 = idx1.astype(jnp.float32)
        r0f = jnp.floor(f0 / W)
        r1f = jnp.floor(f1 / W)
        rows0_o[b, :] = r0f
        cols0_o[b, :] = f0 - r0f * W
        rows1_o[b, :] = r1f
        cols1_o[b, :] = f1 - r1f * W
        per_b.append(dict(desc=desc_loss, kp=kp_loss, prec=precision,
                          rec=recall, sim0=sim0, sim1=sim1,
                          idx0=idx0, idx1=idx1))

    def store_scalar(ref, x):
        ref[...] = jnp.broadcast_to(x, (1, 1))

    store_scalar(desc_o, 0.5 * (per_b[0]["desc"] + per_b[1]["desc"]))
    store_scalar(kp_o, 0.5 * (per_b[0]["kp"] + per_b[1]["kp"]))
    store_scalar(prec_o, 0.5 * (per_b[0]["prec"] + per_b[1]["prec"]))
    store_scalar(rec_o, 0.5 * (per_b[0]["rec"] + per_b[1]["rec"]))

    for b1 in range(B):
        for b2 in range(B):
            g0 = jnp.sum(jnp.where(per_b[b2]["idx0"][:, None] == colK,
                                   per_b[b1]["sim0"][None, :], 0.0), axis=1)
            g1 = jnp.sum(jnp.where(per_b[b2]["idx1"][:, None] == colK,
                                   per_b[b1]["sim1"][None, :], 0.0), axis=1)
            conf_o[b1, b2, :] = g0 * g1


@jax.jit
def kernel(desc_0, desc_1, corr_0, corr_1, logits_0, logits_1):
    d0 = desc_0.reshape(B, C, N)
    d1 = desc_1.reshape(B, C, N)
    c0 = corr_0.reshape(B, 1, N)
    c1 = corr_1.reshape(B, 1, N)
    l0 = logits_0.reshape(B, 1, N)
    l1 = logits_1.reshape(B, 1, N)

    stat = jax.ShapeDtypeStruct((B, 1, N), jnp.float32)
    stat_i = jax.ShapeDtypeStruct((B, 1, N), jnp.int32)
    full = lambda shape: pl.BlockSpec(shape, lambda b, r: (b, 0, 0))
    stats = pl.pallas_call(
        _pass1_kernel,
        grid=(B, NR),
        in_specs=[
            pl.BlockSpec((1, C, TILE_R), lambda b, r: (b, 0, r)),
            pl.BlockSpec((1, C, N), lambda b, r: (b, 0, 0)),
            full((1, 1, N)),
            full((1, 1, N)),
        ],
        out_specs=[full((1, 1, N))] * 8,
        out_shape=[stat, stat, stat_i, stat,
                   stat, stat, stat_i, stat],
    )(d0, d1, c0, c1)

    scal = jax.ShapeDtypeStruct((1, 1), jnp.float32)
    kvec = jax.ShapeDtypeStruct((B, K_PAD), jnp.float32)
    outs = pl.pallas_call(
        _pass2_kernel,
        out_shape=[scal, scal, scal, scal,
                   kvec, kvec, kvec, kvec,
                   jax.ShapeDtypeStruct((B, B, K_PAD), jnp.float32)],
    )(*stats, c0, c1, l0, l1)
    desc_s, kp_s, prec_s, rec_s, rows0, cols0, rows1, cols1, conf = outs

    mp0 = jnp.stack([rows0[:, :K_POINTS], cols0[:, :K_POINTS]], axis=-1)
    mp1 = jnp.stack([rows1[:, :K_POINTS], cols1[:, :K_POINTS]], axis=-1)
    confidence = conf[None, :, :, :K_POINTS]
    return (desc_s[0, 0], kp_s[0, 0], prec_s[0, 0], rec_s[0, 0],
            mp0, mp1, confidence)


# speedup vs baseline: 1.0441x; 1.0227x over previous
import functools
import math

import jax
import jax.numpy as jnp
from jax.experimental import pallas as pl

B, C, H, W = 2, 32, 64, 96
N = H * W
TEMP_SCALE = 1.0 / math.sqrt(0.1)
K_POINTS = 500
K_PAD = 512
TILE_R = 512
NR = N // TILE_R


def _pass1_kernel(d0_ref, d1_ref, c0_ref, c1_ref,
                  rowmax_o, rowsum_o, rowamax_o, v0_o,
                  colmax_o, colsum_o, colamax_o, v1_o):
    r = pl.program_id(1)
    r0 = r * TILE_R
    a = jnp.float32(TEMP_SCALE)
    d0t = d0_ref[0] * a
    d1t = d1_ref[0] * a
    s = jax.lax.dot_general(d0t, d1t, (((0,), (0,)), ((), ())),
                            preferred_element_type=jnp.float32)
    col = jax.lax.broadcasted_iota(jnp.int32, (TILE_R, N), 1)
    row_l = jax.lax.broadcasted_iota(jnp.int32, (TILE_R, N), 0)

    rmax = jnp.max(s, axis=1)
    ramax = jnp.min(jnp.where(s == rmax[:, None], col, N), axis=1)
    rsum = jnp.sum(jnp.exp(s - rmax[:, None]), axis=1)
    c0t = c0_ref[0, 0, pl.ds(r0, TILE_R)]
    v0 = jnp.sum(jnp.where(col == c0t[:, None], s, 0.0), axis=1)
    rowmax_o[0, 0, pl.ds(r0, TILE_R)] = rmax
    rowamax_o[0, 0, pl.ds(r0, TILE_R)] = ramax
    rowsum_o[0, 0, pl.ds(r0, TILE_R)] = rsum
    v0_o[0, 0, pl.ds(r0, TILE_R)] = v0

    @pl.when(r == 0)
    def _init():
        colmax_o[...] = jnp.full((1, 1, N), -jnp.inf, jnp.float32)
        colsum_o[...] = jnp.zeros((1, 1, N), jnp.float32)
        colamax_o[...] = jnp.zeros((1, 1, N), jnp.int32)
        v1_o[...] = jnp.zeros((1, 1, N), jnp.float32)

    old_max = colmax_o[0, 0, :]
    old_sum = colsum_o[0, 0, :]
    old_amax = colamax_o[0, 0, :]
    tmax = jnp.max(s, axis=0)
    tamax = jnp.min(jnp.where(s == tmax[None, :], row_l, TILE_R), axis=0) + r0
    nmax = jnp.maximum(old_max, tmax)
    tsum = jnp.sum(jnp.exp(s - nmax[None, :]), axis=0)
    colmax_o[0, 0, :] = nmax
    colsum_o[0, 0, :] = old_sum * jnp.exp(old_max - nmax) + tsum
    colamax_o[0, 0, :] = jnp.where(tmax > old_max, tamax, old_amax)
    c1v = c1_ref[0, 0, :]
    hit = (c1v[None, :] - r0) == row_l
    v1_o[0, 0, :] += jnp.sum(jnp.where(hit, s, 0.0), axis=0)


def _bce(logits, targets):
    return (jnp.maximum(logits, 0.0) - logits * targets
            + jnp.log1p(jnp.exp(-jnp.abs(logits))))


def _pass2_kernel(rowmax_r, rowsum_r, rowamax_r, v0_r,
                  colmax_r, colsum_r, colamax_r, v1_r,
                  c0_r, c1_r, l0_r, l1_r,
                  desc_o, kp_o, prec_o, rec_o,
                  rows0_o, cols0_o, rows1_o, cols1_o, conf_o):
    colK = jax.lax.broadcasted_iota(jnp.int32, (K_PAD, N), 1)

    def topk_idx(p):
        key2 = jax.lax.bitcast_convert_type(p, jnp.int32) * 2
        colT2 = jax.lax.broadcasted_iota(jnp.int32, (TILE_R, N), 1)
        ranks = []
        for t in range(NR):
            kt = key2[t * TILE_R:(t + 1) * TILE_R]
            gi = t * TILE_R + jax.lax.broadcasted_iota(
                jnp.int32, (TILE_R, N), 0)
            beat = (key2[None, :] + (colT2 < gi).astype(jnp.int32)
                    ) > kt[:, None]
            ranks.append(jnp.sum(beat.astype(jnp.int32), axis=1))
        rank = jnp.concatenate(ranks)
        posK = jax.lax.broadcasted_iota(jnp.int32, (K_PAD, N), 0)
        return jnp.sum(jnp.where(rank[None, :] == posK, colK, 0), axis=1)

    per_b = []
    for b in range(B):
        rowmax = rowmax_r[b, 0, :]
        rowsum = rowsum_r[b, 0, :]
        rowamax = rowamax_r[b, 0, :]
        v0 = v0_r[b, 0, :]
        colmax = colmax_r[b, 0, :]
        colsum = colsum_r[b, 0, :]
        colamax = colamax_r[b, 0, :]
        v1 = v1_r[b, 0, :]
        c0 = c0_r[b, 0, :]
        c1 = c1_r[b, 0, :]
        l0 = l0_r[b, 0, :]
        l1 = l1_r[b, 0, :]

        loss_0 = jnp.mean(-(v0 - rowmax - jnp.log(rowsum)))
        loss_1 = jnp.mean(-(v1 - colmax - jnp.log(colsum)))
        desc_loss = 0.5 * (loss_0 + loss_1)

        cm0 = rowamax == c0
        cm1 = colamax == c1
        iN = jax.lax.broadcasted_iota(jnp.int32, (1, N), 1).reshape(N)
        lhs = colamax * 8192 + iN
        rhs = iN * 8192 + rowamax
        muts = []
        for t in range(NR):
            rt = rhs[t * TILE_R:(t + 1) * TILE_R]
            m = lhs[None, :] == rt[:, None]
            muts.append(jnp.any(m, axis=1).astype(jnp.int32))
        mutual = jnp.concatenate(muts) > 0
        n_correct = jnp.sum((cm0 & mutual).astype(jnp.int32))
        precision = n_correct / jnp.maximum(
            jnp.sum(mutual.astype(jnp.int32)), 1)
        recall = jnp.sum(cm0.astype(jnp.int32)) / N
        kp_loss = 0.5 * (jnp.mean(_bce(l0, cm0.astype(jnp.float32)))
                         + jnp.mean(_bce(l1, cm1.astype(jnp.float32))))

        sim0 = 1.0 / rowsum
        sim1 = 1.0 / colsum
        keep = cm0 & cm1
        p0 = jnp.where(keep, jax.nn.sigmoid(l0), 0.0)
        p1 = jnp.where(keep, jax.nn.sigmoid(l1), 0.0)
        idx0 = topk_idx(p0)
        idx1 = topk_idx(p1)
        f0 = idx0.astype(jnp.float32)
        f1 = idx1.astype(jnp.float32)
        r0f = jnp.floor(f0 / W)
        r1f = jnp.floor(f1 / W)
        rows0_o[b, :] = r0f
        cols0_o[b, :] = f0 - r0f * W
        rows1_o[b, :] = r1f
        cols1_o[b, :] = f1 - r1f * W
        per_b.append(dict(desc=desc_loss, kp=kp_loss, prec=precision,
                          rec=recall, sim0=sim0, sim1=sim1,
                          idx0=idx0, idx1=idx1))

    def store_scalar(ref, x):
        ref[...] = jnp.broadcast_to(x, (1, 1))

    store_scalar(desc_o, 0.5 * (per_b[0]["desc"] + per_b[1]["desc"]))
    store_scalar(kp_o, 0.5 * (per_b[0]["kp"] + per_b[1]["kp"]))
    store_scalar(prec_o, 0.5 * (per_b[0]["prec"] + per_b[1]["prec"]))
    store_scalar(rec_o, 0.5 * (per_b[0]["rec"] + per_b[1]["rec"]))

    for b1 in range(B):
        for b2 in range(B):
            g0 = jnp.sum(jnp.where(per_b[b2]["idx0"][:, None] == colK,
                                   per_b[b1]["sim0"][None, :], 0.0), axis=1)
            g1 = jnp.sum(jnp.where(per_b[b2]["idx1"][:, None] == colK,
                                   per_b[b1]["sim1"][None, :], 0.0), axis=1)
            conf_o[b1, b2, :] = g0 * g1


@jax.jit
def kernel(desc_0, desc_1, corr_0, corr_1, logits_0, logits_1):
    d0 = desc_0.reshape(B, C, N)
    d1 = desc_1.reshape(B, C, N)
    c0 = corr_0.reshape(B, 1, N)
    c1 = corr_1.reshape(B, 1, N)
    l0 = logits_0.reshape(B, 1, N)
    l1 = logits_1.reshape(B, 1, N)

    stat = jax.ShapeDtypeStruct((B, 1, N), jnp.float32)
    stat_i = jax.ShapeDtypeStruct((B, 1, N), jnp.int32)
    full = lambda shape: pl.BlockSpec(shape, lambda b, r: (b, 0, 0))
    stats = pl.pallas_call(
        _pass1_kernel,
        grid=(B, NR),
        in_specs=[
            pl.BlockSpec((1, C, TILE_R), lambda b, r: (b, 0, r)),
            pl.BlockSpec((1, C, N), lambda b, r: (b, 0, 0)),
            full((1, 1, N)),
            full((1, 1, N)),
        ],
        out_specs=[full((1, 1, N))] * 8,
        out_shape=[stat, stat, stat_i, stat,
                   stat, stat, stat_i, stat],
    )(d0, d1, c0, c1)

    scal = jax.ShapeDtypeStruct((1, 1), jnp.float32)
    kvec = jax.ShapeDtypeStruct((B, K_PAD), jnp.float32)
    outs = pl.pallas_call(
        _pass2_kernel,
        out_shape=[scal, scal, scal, scal,
                   kvec, kvec, kvec, kvec,
                   jax.ShapeDtypeStruct((B, B, K_PAD), jnp.float32)],
    )(*stats, c0, c1, l0, l1)
    desc_s, kp_s, prec_s, rec_s, rows0, cols0, rows1, cols1, conf = outs

    mp0 = jnp.stack([rows0[:, :K_POINTS], cols0[:, :K_POINTS]], axis=-1)
    mp1 = jnp.stack([rows1[:, :K_POINTS], cols1[:, :K_POINTS]], axis=-1)
    confidence = conf[None, :, :, :K_POINTS]
    return (desc_s[0, 0], kp_s[0, 0], prec_s[0, 0], rec_s[0, 0],
            mp0, mp1, confidence)
